# baseline (device time: 20379 ns/iter reference)
import jax
import jax.numpy as jnp
from jax import lax
from jax.experimental import pallas as pl
from jax.experimental.pallas import tpu as pltpu

BLK_ROWS = 256


def kernel(x, dy, gamma):
    m, d = x.shape
    grid = m // BLK_ROWS

    def body(x_ref, dy_ref, out_ref, acc_ref, comm_ref, send_sem, recv_sem):
        step = pl.program_id(0)
        my_x = lax.axis_index("x")
        my_y = lax.axis_index("y")
        peer = (my_x, 1 - my_y)

        @pl.when(step == 0)
        def _():
            barrier = pltpu.get_barrier_semaphore()
            pl.semaphore_signal(
                barrier, inc=1, device_id=peer,
                device_id_type=pl.DeviceIdType.MESH,
            )
            pl.semaphore_wait(barrier, 1)
            acc_ref[...] = jnp.zeros_like(acc_ref)

        xb = x_ref[...]
        dyb = dy_ref[...]
        mu = jnp.mean(xb, axis=1, keepdims=True)
        var = jnp.mean((xb - mu) ** 2, axis=1, keepdims=True)
        rstd = lax.rsqrt(var + 1e-5)
        xhat = (xb - mu) * rstd
        dgamma = jnp.sum(dyb * xhat, axis=0, keepdims=True)
        dbeta = jnp.sum(dyb, axis=0, keepdims=True)
        acc_ref[...] += jnp.concatenate([dgamma, dbeta], axis=0)

        @pl.when(step == grid - 1)
        def _():
            rdma = pltpu.make_async_remote_copy(
                src_ref=acc_ref,
                dst_ref=comm_ref,
                send_sem=send_sem,
                recv_sem=recv_sem,
                device_id=peer,
                device_id_type=pl.DeviceIdType.MESH,
            )
            rdma.start()
            rdma.wait()
            out_ref[...] = acc_ref[...] + comm_ref[...]

    return pl.pallas_call(
        body,
        grid=(grid,),
        out_shape=jax.ShapeDtypeStruct((2, d), jnp.float32),
        in_specs=[
            pl.BlockSpec((BLK_ROWS, d), lambda i: (i, 0)),
            pl.BlockSpec((BLK_ROWS, d), lambda i: (i, 0)),
        ],
        out_specs=pl.BlockSpec((2, d), lambda i: (0, 0)),
        scratch_shapes=[
            pltpu.VMEM((2, d), jnp.float32),
            pltpu.VMEM((2, d), jnp.float32),
            pltpu.SemaphoreType.DMA,
            pltpu.SemaphoreType.DMA,
        ],
        compiler_params=pltpu.CompilerParams(collective_id=0),
    )(x, dy)


# device time: 18683 ns/iter; 1.0908x vs baseline; 1.0908x over previous
import jax
import jax.numpy as jnp
from jax import lax
from jax.experimental import pallas as pl
from jax.experimental.pallas import tpu as pltpu

BLK_ROWS = 512


def kernel(x, dy, gamma):
    m, d = x.shape
    grid = m // BLK_ROWS

    def body(x_ref, dy_ref, out_ref, acc_ref, comm_ref, send_sem, recv_sem):
        step = pl.program_id(0)
        my_x = lax.axis_index("x")
        my_y = lax.axis_index("y")
        peer = (my_x, 1 - my_y)

        @pl.when(step == 0)
        def _():
            barrier = pltpu.get_barrier_semaphore()
            pl.semaphore_signal(
                barrier, inc=1, device_id=peer,
                device_id_type=pl.DeviceIdType.MESH,
            )
            pl.semaphore_wait(barrier, 1)
            acc_ref[...] = jnp.zeros_like(acc_ref)

        xb = x_ref[...]
        dyb = dy_ref[...]
        mu = jnp.mean(xb, axis=1, keepdims=True)
        ex2 = jnp.mean(xb * xb, axis=1, keepdims=True)
        rstd = lax.rsqrt(ex2 - mu * mu + 1e-5)
        dgamma = jnp.sum(dyb * (xb * rstd - mu * rstd), axis=0, keepdims=True)
        dbeta = jnp.sum(dyb, axis=0, keepdims=True)
        acc_ref[...] += jnp.concatenate([dgamma, dbeta], axis=0)

        @pl.when(step == grid - 1)
        def _():
            rdma = pltpu.make_async_remote_copy(
                src_ref=acc_ref,
                dst_ref=comm_ref,
                send_sem=send_sem,
                recv_sem=recv_sem,
                device_id=peer,
                device_id_type=pl.DeviceIdType.MESH,
            )
            rdma.start()
            rdma.wait()
            out_ref[...] = acc_ref[...] + comm_ref[...]

    return pl.pallas_call(
        body,
        grid=(grid,),
        out_shape=jax.ShapeDtypeStruct((2, d), jnp.float32),
        in_specs=[
            pl.BlockSpec((BLK_ROWS, d), lambda i: (i, 0)),
            pl.BlockSpec((BLK_ROWS, d), lambda i: (i, 0)),
        ],
        out_specs=pl.BlockSpec((2, d), lambda i: (0, 0)),
        scratch_shapes=[
            pltpu.VMEM((2, d), jnp.float32),
            pltpu.VMEM((2, d), jnp.float32),
            pltpu.SemaphoreType.DMA,
            pltpu.SemaphoreType.DMA,
        ],
        compiler_params=pltpu.CompilerParams(collective_id=0),
    )(x, dy)
